# initial kernel scaffold (unmeasured)
import jax
import jax.numpy as jnp
from jax import lax
from jax.experimental import pallas as pl
from jax.experimental.pallas import tpu as pltpu


def kernel(
    x,
):
    def body(*refs):
        pass

    out_shape = jax.ShapeDtypeStruct(..., jnp.float32)
    return pl.pallas_call(body, out_shape=out_shape)(...)


# baseline (device time: 492088 ns/iter reference)
import jax
import jax.numpy as jnp
from jax import lax
from jax.experimental import pallas as pl
from jax.experimental.pallas import tpu as pltpu

M = 16384
N = 2048
NH = 1024
C = 1024
NC = M // C
NSTEPS = NC // 2


def kernel(x):
    def body(x_ref, out_ref, xs_f32, xl_f32, send_buf, recv_x, recv_y, acc,
             load_sem, send_sem_x, recv_sem_x, send_sem_y, recv_sem_y,
             out_sem):
        my_x = lax.axis_index("x")
        my_y = lax.axis_index("y")
        my_z = lax.axis_index("z")
        peer = (1 - my_x, my_y, my_z)
        ynbr = (my_x, 1 - my_y, my_z)
        my_col = my_x * NH
        peer_col = (1 - my_x) * NH

        barrier = pltpu.get_barrier_semaphore()
        for nbr in (peer, ynbr):
            pl.semaphore_signal(barrier, inc=1, device_id=nbr,
                                device_id_type=pl.DeviceIdType.MESH)
        pl.semaphore_wait(barrier, 2)

        for s in range(NSTEPS):
            slot = s % 2
            c_d = 2 * s + my_y
            c_r = 2 * s + (1 - my_y)

            ld = pltpu.make_async_copy(
                x_ref.at[0, pl.ds(c_d * C, C), pl.ds(peer_col, NH)],
                xs_f32.at[slot], load_sem.at[0])
            ld.start()
            ld.wait()
            send_buf[slot] = xs_f32[slot].astype(jnp.bfloat16)

            rdma_x = pltpu.make_async_remote_copy(
                src_ref=send_buf.at[slot], dst_ref=recv_x.at[slot],
                send_sem=send_sem_x.at[slot], recv_sem=recv_sem_x.at[slot],
                device_id=peer, device_id_type=pl.DeviceIdType.MESH)
            rdma_x.start()
            rdma_x.wait()

            rdma_y = pltpu.make_async_remote_copy(
                src_ref=recv_x.at[slot], dst_ref=recv_y.at[slot],
                send_sem=send_sem_y.at[slot], recv_sem=recv_sem_y.at[slot],
                device_id=ynbr, device_id_type=pl.DeviceIdType.MESH)
            rdma_y.start()
            rdma_y.wait()

            for c, rbuf in ((c_d, recv_x), (c_r, recv_y)):
                ll = pltpu.make_async_copy(
                    x_ref.at[0, pl.ds(c * C, C), pl.ds(my_col, NH)],
                    xl_f32.at[0], load_sem.at[1])
                ll.start()
                ll.wait()
                acc[0] = xl_f32[0].astype(jnp.bfloat16) + rbuf[slot]
                st = pltpu.make_async_copy(
                    acc.at[0], out_ref.at[pl.ds(c * C, C), :], out_sem.at[0])
                st.start()
                st.wait()

    return pl.pallas_call(
        body,
        out_shape=jax.ShapeDtypeStruct((M, NH), jnp.bfloat16),
        in_specs=[pl.BlockSpec(memory_space=pl.ANY)],
        out_specs=pl.BlockSpec(memory_space=pl.ANY),
        scratch_shapes=[
            pltpu.VMEM((2, C, NH), jnp.float32),
            pltpu.VMEM((1, C, NH), jnp.float32),
            pltpu.VMEM((2, C, NH), jnp.bfloat16),
            pltpu.VMEM((2, C, NH), jnp.bfloat16),
            pltpu.VMEM((2, C, NH), jnp.bfloat16),
            pltpu.VMEM((1, C, NH), jnp.bfloat16),
            pltpu.SemaphoreType.DMA((2,)),
            pltpu.SemaphoreType.DMA((2,)),
            pltpu.SemaphoreType.DMA((2,)),
            pltpu.SemaphoreType.DMA((2,)),
            pltpu.SemaphoreType.DMA((2,)),
            pltpu.SemaphoreType.DMA((1,)),
        ],
        compiler_params=pltpu.CompilerParams(collective_id=0),
    )(x)


# device time: 222065 ns/iter; 2.2160x vs baseline; 2.2160x over previous
import jax
import jax.numpy as jnp
from jax import lax
from jax.experimental import pallas as pl
from jax.experimental.pallas import tpu as pltpu

M = 16384
N = 2048
NH = 1024
C = 512
NC = M // C
NS = NC // 2
KS = 6
SS = 4
CREDIT_LAG = 3


def kernel(x):
    def body(x_ref, out_ref, xs_f32, xld_f32, xlr_f32, send_buf, recv_x,
             recv_y, acc_d, acc_r, ls_sem, ld_sem, lr_sem, od_sem, or_sem,
             send_sem_x, recv_sem_x, send_sem_y, recv_sem_y,
             credit_x, credit_y):
        my_x = lax.axis_index("x")
        my_y = lax.axis_index("y")
        my_z = lax.axis_index("z")
        peer = (1 - my_x, my_y, my_z)
        ynbr = (my_x, 1 - my_y, my_z)
        my_col = my_x * NH
        peer_col = (1 - my_x) * NH

        def c_dir(s):
            return 2 * s + my_y

        def c_rel(s):
            return 2 * s + (1 - my_y)

        def load(cols, c, dst, sem):
            cp = pltpu.make_async_copy(
                x_ref.at[0, pl.ds(c * C, C), pl.ds(cols, NH)], dst, sem)
            cp.start()
            return cp

        def rdma_dir(s):
            return pltpu.make_async_remote_copy(
                src_ref=send_buf.at[s % SS], dst_ref=recv_x.at[s % KS],
                send_sem=send_sem_x.at[s % KS], recv_sem=recv_sem_x.at[s % KS],
                device_id=peer, device_id_type=pl.DeviceIdType.MESH)

        def rdma_rel(s):
            return pltpu.make_async_remote_copy(
                src_ref=recv_x.at[s % KS], dst_ref=recv_y.at[s % KS],
                send_sem=send_sem_y.at[s % KS], recv_sem=recv_sem_y.at[s % KS],
                device_id=ynbr, device_id_type=pl.DeviceIdType.MESH)

        barrier = pltpu.get_barrier_semaphore()
        for nbr in (peer, ynbr):
            pl.semaphore_signal(barrier, inc=1, device_id=nbr,
                                device_id_type=pl.DeviceIdType.MESH)
        pl.semaphore_wait(barrier, 2)

        descs_x = [None] * NS
        descs_y = [None] * NS
        loads_s = [None] * NS
        loads_d = [None] * NS
        loads_r = [None] * NS
        outs_d = [None] * NS
        outs_r = [None] * NS

        loads_s[0] = load(peer_col, c_dir(0), xs_f32.at[0], ls_sem.at[0])
        loads_s[1] = load(peer_col, c_dir(1), xs_f32.at[1], ls_sem.at[1])
        loads_d[0] = load(my_col, c_dir(0), xld_f32.at[0], ld_sem.at[0])
        loads_r[0] = load(my_col, c_rel(0), xlr_f32.at[0], lr_sem.at[0])
        for v in (0, 1):
            loads_s[v].wait()
            send_buf[v % SS] = xs_f32[v % 2].astype(jnp.bfloat16)
            loads_s[v + 2] = load(peer_col, c_dir(v + 2), xs_f32.at[v % 2],
                                  ls_sem.at[v % 2])
            descs_x[v] = rdma_dir(v)
            descs_x[v].start()

        for s in range(NS):
            descs_x[s].wait_recv()
            if s >= KS:
                pl.semaphore_wait(credit_y, 1)
            descs_y[s] = rdma_rel(s)
            descs_y[s].start()

            v = s + 2
            if v < NS:
                loads_s[v].wait()
                if v >= SS:
                    descs_x[v - SS].wait_send()
                send_buf[v % SS] = xs_f32[v % 2].astype(jnp.bfloat16)
                if v + 2 < NS:
                    loads_s[v + 2] = load(peer_col, c_dir(v + 2),
                                          xs_f32.at[v % 2], ls_sem.at[v % 2])
                if v >= KS:
                    pl.semaphore_wait(credit_x, 1)
                descs_x[v] = rdma_dir(v)
                descs_x[v].start()

            loads_d[s].wait()
            if s + 1 < NS:
                loads_d[s + 1] = load(my_col, c_dir(s + 1),
                                      xld_f32.at[(s + 1) % 2],
                                      ld_sem.at[(s + 1) % 2])
            if s >= 2:
                outs_d[s - 2].wait()
            acc_d[s % 2] = xld_f32[s % 2].astype(jnp.bfloat16) + recv_x[s % KS]
            outs_d[s] = pltpu.make_async_copy(
                acc_d.at[s % 2], out_ref.at[pl.ds(c_dir(s) * C, C), :],
                od_sem.at[s % 2])
            outs_d[s].start()

            if s >= 1:
                t = s - 1
                descs_y[t].wait_recv()
                loads_r[t].wait()
                if t + 1 < NS:
                    loads_r[t + 1] = load(my_col, c_rel(t + 1),
                                          xlr_f32.at[(t + 1) % 2],
                                          lr_sem.at[(t + 1) % 2])
                if t >= 2:
                    outs_r[t - 2].wait()
                acc_r[t % 2] = (xlr_f32[t % 2].astype(jnp.bfloat16)
                                + recv_y[t % KS])
                outs_r[t] = pltpu.make_async_copy(
                    acc_r.at[t % 2], out_ref.at[pl.ds(c_rel(t) * C, C), :],
                    or_sem.at[t % 2])
                outs_r[t].start()
                if t + KS < NS:
                    pl.semaphore_signal(credit_y, inc=1, device_id=ynbr,
                                        device_id_type=pl.DeviceIdType.MESH)

            if s >= CREDIT_LAG:
                u = s - CREDIT_LAG
                descs_y[u].wait_send()
                if u + KS < NS:
                    pl.semaphore_signal(credit_x, inc=1, device_id=peer,
                                        device_id_type=pl.DeviceIdType.MESH)

        t = NS - 1
        descs_y[t].wait_recv()
        loads_r[t].wait()
        outs_r[t - 2].wait()
        acc_r[t % 2] = xlr_f32[t % 2].astype(jnp.bfloat16) + recv_y[t % KS]
        outs_r[t] = pltpu.make_async_copy(
            acc_r.at[t % 2], out_ref.at[pl.ds(c_rel(t) * C, C), :],
            or_sem.at[t % 2])
        outs_r[t].start()

        for v in range(NS - SS, NS):
            descs_x[v].wait_send()
        for u in range(NS - CREDIT_LAG, NS):
            descs_y[u].wait_send()
        for w in range(NS - 2, NS):
            outs_d[w].wait()
            outs_r[w].wait()

    return pl.pallas_call(
        body,
        out_shape=jax.ShapeDtypeStruct((M, NH), jnp.bfloat16),
        in_specs=[pl.BlockSpec(memory_space=pl.ANY)],
        out_specs=pl.BlockSpec(memory_space=pl.ANY),
        scratch_shapes=[
            pltpu.VMEM((2, C, NH), jnp.float32),
            pltpu.VMEM((2, C, NH), jnp.float32),
            pltpu.VMEM((2, C, NH), jnp.float32),
            pltpu.VMEM((SS, C, NH), jnp.bfloat16),
            pltpu.VMEM((KS, C, NH), jnp.bfloat16),
            pltpu.VMEM((KS, C, NH), jnp.bfloat16),
            pltpu.VMEM((2, C, NH), jnp.bfloat16),
            pltpu.VMEM((2, C, NH), jnp.bfloat16),
            pltpu.SemaphoreType.DMA((2,)),
            pltpu.SemaphoreType.DMA((2,)),
            pltpu.SemaphoreType.DMA((2,)),
            pltpu.SemaphoreType.DMA((2,)),
            pltpu.SemaphoreType.DMA((2,)),
            pltpu.SemaphoreType.DMA((KS,)),
            pltpu.SemaphoreType.DMA((KS,)),
            pltpu.SemaphoreType.DMA((KS,)),
            pltpu.SemaphoreType.DMA((KS,)),
            pltpu.SemaphoreType.REGULAR,
            pltpu.SemaphoreType.REGULAR,
        ],
        compiler_params=pltpu.CompilerParams(
            collective_id=0, vmem_limit_bytes=48 * 1024 * 1024),
    )(x)


# device time: 222042 ns/iter; 2.2162x vs baseline; 1.0001x over previous
import jax
import jax.numpy as jnp
from jax import lax
from jax.experimental import pallas as pl
from jax.experimental.pallas import tpu as pltpu

M = 16384
N = 2048
NH = 1024
C = 512
NC = M // C
NS = NC // 2
KS = 8
SS = 4
CREDIT_LAG = 3


def kernel(x):
    def body(x_ref, out_ref, xs_f32, xld_f32, xlr_f32, send_buf, recv_x,
             recv_y, acc_d, acc_r, ls_sem, ld_sem, lr_sem, od_sem, or_sem,
             send_sem_x, recv_sem_x, send_sem_y, recv_sem_y,
             credit_x, credit_y):
        my_x = lax.axis_index("x")
        my_y = lax.axis_index("y")
        my_z = lax.axis_index("z")
        peer = (1 - my_x, my_y, my_z)
        ynbr = (my_x, 1 - my_y, my_z)
        my_col = my_x * NH
        peer_col = (1 - my_x) * NH

        def c_dir(s):
            return 2 * s + my_y

        def c_rel(s):
            return 2 * s + (1 - my_y)

        def load(cols, c, dst, sem):
            cp = pltpu.make_async_copy(
                x_ref.at[0, pl.ds(c * C, C), pl.ds(cols, NH)], dst, sem)
            cp.start()
            return cp

        def rdma_dir(s):
            return pltpu.make_async_remote_copy(
                src_ref=send_buf.at[s % SS], dst_ref=recv_x.at[s % KS],
                send_sem=send_sem_x.at[s % KS], recv_sem=recv_sem_x.at[s % KS],
                device_id=peer, device_id_type=pl.DeviceIdType.MESH)

        def rdma_rel(s):
            return pltpu.make_async_remote_copy(
                src_ref=recv_x.at[s % KS], dst_ref=recv_y.at[s % KS],
                send_sem=send_sem_y.at[s % KS], recv_sem=recv_sem_y.at[s % KS],
                device_id=ynbr, device_id_type=pl.DeviceIdType.MESH)

        barrier = pltpu.get_barrier_semaphore()
        for nbr in (peer, ynbr):
            pl.semaphore_signal(barrier, inc=1, device_id=nbr,
                                device_id_type=pl.DeviceIdType.MESH)
        pl.semaphore_wait(barrier, 2)

        descs_x = [None] * NS
        descs_y = [None] * NS
        loads_s = [None] * NS
        loads_d = [None] * NS
        loads_r = [None] * NS
        outs_d = [None] * NS
        outs_r = [None] * NS

        loads_s[0] = load(peer_col, c_dir(0), xs_f32.at[0], ls_sem.at[0])
        loads_s[1] = load(peer_col, c_dir(1), xs_f32.at[1], ls_sem.at[1])
        loads_d[0] = load(my_col, c_dir(0), xld_f32.at[0], ld_sem.at[0])
        loads_r[0] = load(my_col, c_rel(0), xlr_f32.at[0], lr_sem.at[0])
        for v in (0, 1, 2):
            loads_s[v].wait()
            send_buf[v % SS] = xs_f32[v % 2].astype(jnp.bfloat16)
            loads_s[v + 2] = load(peer_col, c_dir(v + 2), xs_f32.at[v % 2],
                                  ls_sem.at[v % 2])
            descs_x[v] = rdma_dir(v)
            descs_x[v].start()

        for s in range(NS):
            descs_x[s].wait_recv()
            if s >= KS:
                pl.semaphore_wait(credit_y, 1)
            descs_y[s] = rdma_rel(s)
            descs_y[s].start()

            v = s + 3
            if v < NS:
                loads_s[v].wait()
                if v >= SS:
                    descs_x[v - SS].wait_send()
                send_buf[v % SS] = xs_f32[v % 2].astype(jnp.bfloat16)
                if v + 2 < NS:
                    loads_s[v + 2] = load(peer_col, c_dir(v + 2),
                                          xs_f32.at[v % 2], ls_sem.at[v % 2])
                if v >= KS:
                    pl.semaphore_wait(credit_x, 1)
                descs_x[v] = rdma_dir(v)
                descs_x[v].start()

            loads_d[s].wait()
            if s + 1 < NS:
                loads_d[s + 1] = load(my_col, c_dir(s + 1),
                                      xld_f32.at[(s + 1) % 2],
                                      ld_sem.at[(s + 1) % 2])
            if s >= 2:
                outs_d[s - 2].wait()
            acc_d[s % 2] = xld_f32[s % 2].astype(jnp.bfloat16) + recv_x[s % KS]
            outs_d[s] = pltpu.make_async_copy(
                acc_d.at[s % 2], out_ref.at[pl.ds(c_dir(s) * C, C), :],
                od_sem.at[s % 2])
            outs_d[s].start()

            if s >= 1:
                t = s - 1
                descs_y[t].wait_recv()
                loads_r[t].wait()
                if t + 1 < NS:
                    loads_r[t + 1] = load(my_col, c_rel(t + 1),
                                          xlr_f32.at[(t + 1) % 2],
                                          lr_sem.at[(t + 1) % 2])
                if t >= 2:
                    outs_r[t - 2].wait()
                acc_r[t % 2] = (xlr_f32[t % 2].astype(jnp.bfloat16)
                                + recv_y[t % KS])
                outs_r[t] = pltpu.make_async_copy(
                    acc_r.at[t % 2], out_ref.at[pl.ds(c_rel(t) * C, C), :],
                    or_sem.at[t % 2])
                outs_r[t].start()
                if t + KS < NS:
                    pl.semaphore_signal(credit_y, inc=1, device_id=ynbr,
                                        device_id_type=pl.DeviceIdType.MESH)

            if s >= CREDIT_LAG:
                u = s - CREDIT_LAG
                descs_y[u].wait_send()
                if u + KS < NS:
                    pl.semaphore_signal(credit_x, inc=1, device_id=peer,
                                        device_id_type=pl.DeviceIdType.MESH)

        t = NS - 1
        descs_y[t].wait_recv()
        loads_r[t].wait()
        outs_r[t - 2].wait()
        acc_r[t % 2] = xlr_f32[t % 2].astype(jnp.bfloat16) + recv_y[t % KS]
        outs_r[t] = pltpu.make_async_copy(
            acc_r.at[t % 2], out_ref.at[pl.ds(c_rel(t) * C, C), :],
            or_sem.at[t % 2])
        outs_r[t].start()

        for v in range(NS - SS, NS):
            descs_x[v].wait_send()
        for u in range(NS - CREDIT_LAG, NS):
            descs_y[u].wait_send()
        for w in range(NS - 2, NS):
            outs_d[w].wait()
            outs_r[w].wait()

    return pl.pallas_call(
        body,
        out_shape=jax.ShapeDtypeStruct((M, NH), jnp.bfloat16),
        in_specs=[pl.BlockSpec(memory_space=pl.ANY)],
        out_specs=pl.BlockSpec(memory_space=pl.ANY),
        scratch_shapes=[
            pltpu.VMEM((2, C, NH), jnp.float32),
            pltpu.VMEM((2, C, NH), jnp.float32),
            pltpu.VMEM((2, C, NH), jnp.float32),
            pltpu.VMEM((SS, C, NH), jnp.bfloat16),
            pltpu.VMEM((KS, C, NH), jnp.bfloat16),
            pltpu.VMEM((KS, C, NH), jnp.bfloat16),
            pltpu.VMEM((2, C, NH), jnp.bfloat16),
            pltpu.VMEM((2, C, NH), jnp.bfloat16),
            pltpu.SemaphoreType.DMA((2,)),
            pltpu.SemaphoreType.DMA((2,)),
            pltpu.SemaphoreType.DMA((2,)),
            pltpu.SemaphoreType.DMA((2,)),
            pltpu.SemaphoreType.DMA((2,)),
            pltpu.SemaphoreType.DMA((KS,)),
            pltpu.SemaphoreType.DMA((KS,)),
            pltpu.SemaphoreType.DMA((KS,)),
            pltpu.SemaphoreType.DMA((KS,)),
            pltpu.SemaphoreType.REGULAR,
            pltpu.SemaphoreType.REGULAR,
        ],
        compiler_params=pltpu.CompilerParams(
            collective_id=0, vmem_limit_bytes=48 * 1024 * 1024),
    )(x)


# device time: 216001 ns/iter; 2.2782x vs baseline; 1.0280x over previous
import jax
import jax.numpy as jnp
from jax import lax
from jax.experimental import pallas as pl
from jax.experimental.pallas import tpu as pltpu

M = 16384
N = 2048
NH = 1024
C = 256
NC = M // C
NS = NC // 2
KS = 8
SS = 4
CREDIT_LAG = 3


def kernel(x):
    def body(x_ref, out_ref, xs_f32, xld_f32, xlr_f32, send_buf, recv_x,
             recv_y, acc_d, acc_r, ls_sem, ld_sem, lr_sem, od_sem, or_sem,
             send_sem_x, recv_sem_x, send_sem_y, recv_sem_y,
             credit_x, credit_y):
        my_x = lax.axis_index("x")
        my_y = lax.axis_index("y")
        my_z = lax.axis_index("z")
        peer = (1 - my_x, my_y, my_z)
        ynbr = (my_x, 1 - my_y, my_z)
        my_col = my_x * NH
        peer_col = (1 - my_x) * NH

        def c_dir(s):
            return 2 * s + my_y

        def c_rel(s):
            return 2 * s + (1 - my_y)

        def load(cols, c, dst, sem):
            cp = pltpu.make_async_copy(
                x_ref.at[0, pl.ds(c * C, C), pl.ds(cols, NH)], dst, sem)
            cp.start()
            return cp

        def rdma_dir(s):
            return pltpu.make_async_remote_copy(
                src_ref=send_buf.at[s % SS], dst_ref=recv_x.at[s % KS],
                send_sem=send_sem_x.at[s % KS], recv_sem=recv_sem_x.at[s % KS],
                device_id=peer, device_id_type=pl.DeviceIdType.MESH)

        def rdma_rel(s):
            return pltpu.make_async_remote_copy(
                src_ref=recv_x.at[s % KS], dst_ref=recv_y.at[s % KS],
                send_sem=send_sem_y.at[s % KS], recv_sem=recv_sem_y.at[s % KS],
                device_id=ynbr, device_id_type=pl.DeviceIdType.MESH)

        barrier = pltpu.get_barrier_semaphore()
        for nbr in (peer, ynbr):
            pl.semaphore_signal(barrier, inc=1, device_id=nbr,
                                device_id_type=pl.DeviceIdType.MESH)
        pl.semaphore_wait(barrier, 2)

        descs_x = [None] * NS
        descs_y = [None] * NS
        loads_s = [None] * NS
        loads_d = [None] * NS
        loads_r = [None] * NS
        outs_d = [None] * NS
        outs_r = [None] * NS

        loads_s[0] = load(peer_col, c_dir(0), xs_f32.at[0], ls_sem.at[0])
        loads_s[1] = load(peer_col, c_dir(1), xs_f32.at[1], ls_sem.at[1])
        loads_d[0] = load(my_col, c_dir(0), xld_f32.at[0], ld_sem.at[0])
        loads_r[0] = load(my_col, c_rel(0), xlr_f32.at[0], lr_sem.at[0])
        for v in (0, 1, 2):
            loads_s[v].wait()
            send_buf[v % SS] = xs_f32[v % 2].astype(jnp.bfloat16)
            loads_s[v + 2] = load(peer_col, c_dir(v + 2), xs_f32.at[v % 2],
                                  ls_sem.at[v % 2])
            descs_x[v] = rdma_dir(v)
            descs_x[v].start()

        for s in range(NS):
            descs_x[s].wait_recv()
            if s >= KS:
                pl.semaphore_wait(credit_y, 1)
            descs_y[s] = rdma_rel(s)
            descs_y[s].start()

            v = s + 3
            if v < NS:
                loads_s[v].wait()
                if v >= SS:
                    descs_x[v - SS].wait_send()
                send_buf[v % SS] = xs_f32[v % 2].astype(jnp.bfloat16)
                if v + 2 < NS:
                    loads_s[v + 2] = load(peer_col, c_dir(v + 2),
                                          xs_f32.at[v % 2], ls_sem.at[v % 2])
                if v >= KS:
                    pl.semaphore_wait(credit_x, 1)
                descs_x[v] = rdma_dir(v)
                descs_x[v].start()

            loads_d[s].wait()
            if s + 1 < NS:
                loads_d[s + 1] = load(my_col, c_dir(s + 1),
                                      xld_f32.at[(s + 1) % 2],
                                      ld_sem.at[(s + 1) % 2])
            if s >= 2:
                outs_d[s - 2].wait()
            acc_d[s % 2] = xld_f32[s % 2].astype(jnp.bfloat16) + recv_x[s % KS]
            outs_d[s] = pltpu.make_async_copy(
                acc_d.at[s % 2], out_ref.at[pl.ds(c_dir(s) * C, C), :],
                od_sem.at[s % 2])
            outs_d[s].start()

            if s >= 1:
                t = s - 1
                descs_y[t].wait_recv()
                loads_r[t].wait()
                if t + 1 < NS:
                    loads_r[t + 1] = load(my_col, c_rel(t + 1),
                                          xlr_f32.at[(t + 1) % 2],
                                          lr_sem.at[(t + 1) % 2])
                if t >= 2:
                    outs_r[t - 2].wait()
                acc_r[t % 2] = (xlr_f32[t % 2].astype(jnp.bfloat16)
                                + recv_y[t % KS])
                outs_r[t] = pltpu.make_async_copy(
                    acc_r.at[t % 2], out_ref.at[pl.ds(c_rel(t) * C, C), :],
                    or_sem.at[t % 2])
                outs_r[t].start()
                if t + KS < NS:
                    pl.semaphore_signal(credit_y, inc=1, device_id=ynbr,
                                        device_id_type=pl.DeviceIdType.MESH)

            if s >= CREDIT_LAG:
                u = s - CREDIT_LAG
                descs_y[u].wait_send()
                if u + KS < NS:
                    pl.semaphore_signal(credit_x, inc=1, device_id=peer,
                                        device_id_type=pl.DeviceIdType.MESH)

        t = NS - 1
        descs_y[t].wait_recv()
        loads_r[t].wait()
        outs_r[t - 2].wait()
        acc_r[t % 2] = xlr_f32[t % 2].astype(jnp.bfloat16) + recv_y[t % KS]
        outs_r[t] = pltpu.make_async_copy(
            acc_r.at[t % 2], out_ref.at[pl.ds(c_rel(t) * C, C), :],
            or_sem.at[t % 2])
        outs_r[t].start()

        for v in range(NS - SS, NS):
            descs_x[v].wait_send()
        for u in range(NS - CREDIT_LAG, NS):
            descs_y[u].wait_send()
        for w in range(NS - 2, NS):
            outs_d[w].wait()
            outs_r[w].wait()

    return pl.pallas_call(
        body,
        out_shape=jax.ShapeDtypeStruct((M, NH), jnp.bfloat16),
        in_specs=[pl.BlockSpec(memory_space=pl.ANY)],
        out_specs=pl.BlockSpec(memory_space=pl.ANY),
        scratch_shapes=[
            pltpu.VMEM((2, C, NH), jnp.float32),
            pltpu.VMEM((2, C, NH), jnp.float32),
            pltpu.VMEM((2, C, NH), jnp.float32),
            pltpu.VMEM((SS, C, NH), jnp.bfloat16),
            pltpu.VMEM((KS, C, NH), jnp.bfloat16),
            pltpu.VMEM((KS, C, NH), jnp.bfloat16),
            pltpu.VMEM((2, C, NH), jnp.bfloat16),
            pltpu.VMEM((2, C, NH), jnp.bfloat16),
            pltpu.SemaphoreType.DMA((2,)),
            pltpu.SemaphoreType.DMA((2,)),
            pltpu.SemaphoreType.DMA((2,)),
            pltpu.SemaphoreType.DMA((2,)),
            pltpu.SemaphoreType.DMA((2,)),
            pltpu.SemaphoreType.DMA((KS,)),
            pltpu.SemaphoreType.DMA((KS,)),
            pltpu.SemaphoreType.DMA((KS,)),
            pltpu.SemaphoreType.DMA((KS,)),
            pltpu.SemaphoreType.REGULAR,
            pltpu.SemaphoreType.REGULAR,
        ],
        compiler_params=pltpu.CompilerParams(
            collective_id=0, vmem_limit_bytes=48 * 1024 * 1024),
    )(x)


# device time: 213091 ns/iter; 2.3093x vs baseline; 1.0137x over previous
import jax
import jax.numpy as jnp
from jax import lax
from jax.experimental import pallas as pl
from jax.experimental.pallas import tpu as pltpu

M = 16384
N = 2048
NH = 1024
C = 128
NC = M // C
NS = NC // 2
KS = 8
SS = 4
CREDIT_LAG = 3


def kernel(x):
    def body(x_ref, out_ref, xs_f32, xld_f32, xlr_f32, send_buf, recv_x,
             recv_y, acc_d, acc_r, ls_sem, ld_sem, lr_sem, od_sem, or_sem,
             send_sem_x, recv_sem_x, send_sem_y, recv_sem_y,
             credit_x, credit_y):
        my_x = lax.axis_index("x")
        my_y = lax.axis_index("y")
        my_z = lax.axis_index("z")
        peer = (1 - my_x, my_y, my_z)
        ynbr = (my_x, 1 - my_y, my_z)
        my_col = my_x * NH
        peer_col = (1 - my_x) * NH

        def c_dir(s):
            return 2 * s + my_y

        def c_rel(s):
            return 2 * s + (1 - my_y)

        def load(cols, c, dst, sem):
            cp = pltpu.make_async_copy(
                x_ref.at[0, pl.ds(c * C, C), pl.ds(cols, NH)], dst, sem)
            cp.start()
            return cp

        def rdma_dir(s):
            return pltpu.make_async_remote_copy(
                src_ref=send_buf.at[s % SS], dst_ref=recv_x.at[s % KS],
                send_sem=send_sem_x.at[s % KS], recv_sem=recv_sem_x.at[s % KS],
                device_id=peer, device_id_type=pl.DeviceIdType.MESH)

        def rdma_rel(s):
            return pltpu.make_async_remote_copy(
                src_ref=recv_x.at[s % KS], dst_ref=recv_y.at[s % KS],
                send_sem=send_sem_y.at[s % KS], recv_sem=recv_sem_y.at[s % KS],
                device_id=ynbr, device_id_type=pl.DeviceIdType.MESH)

        barrier = pltpu.get_barrier_semaphore()
        for nbr in (peer, ynbr):
            pl.semaphore_signal(barrier, inc=1, device_id=nbr,
                                device_id_type=pl.DeviceIdType.MESH)
        pl.semaphore_wait(barrier, 2)

        descs_x = [None] * NS
        descs_y = [None] * NS
        loads_s = [None] * NS
        loads_d = [None] * NS
        loads_r = [None] * NS
        outs_d = [None] * NS
        outs_r = [None] * NS

        loads_s[0] = load(peer_col, c_dir(0), xs_f32.at[0], ls_sem.at[0])
        loads_s[1] = load(peer_col, c_dir(1), xs_f32.at[1], ls_sem.at[1])
        loads_d[0] = load(my_col, c_dir(0), xld_f32.at[0], ld_sem.at[0])
        loads_r[0] = load(my_col, c_rel(0), xlr_f32.at[0], lr_sem.at[0])
        for v in (0, 1, 2):
            loads_s[v].wait()
            send_buf[v % SS] = xs_f32[v % 2].astype(jnp.bfloat16)
            loads_s[v + 2] = load(peer_col, c_dir(v + 2), xs_f32.at[v % 2],
                                  ls_sem.at[v % 2])
            descs_x[v] = rdma_dir(v)
            descs_x[v].start()

        for s in range(NS):
            descs_x[s].wait_recv()
            if s >= KS:
                pl.semaphore_wait(credit_y, 1)
            descs_y[s] = rdma_rel(s)
            descs_y[s].start()

            v = s + 3
            if v < NS:
                loads_s[v].wait()
                if v >= SS:
                    descs_x[v - SS].wait_send()
                send_buf[v % SS] = xs_f32[v % 2].astype(jnp.bfloat16)
                if v + 2 < NS:
                    loads_s[v + 2] = load(peer_col, c_dir(v + 2),
                                          xs_f32.at[v % 2], ls_sem.at[v % 2])
                if v >= KS:
                    pl.semaphore_wait(credit_x, 1)
                descs_x[v] = rdma_dir(v)
                descs_x[v].start()

            loads_d[s].wait()
            if s + 1 < NS:
                loads_d[s + 1] = load(my_col, c_dir(s + 1),
                                      xld_f32.at[(s + 1) % 2],
                                      ld_sem.at[(s + 1) % 2])
            if s >= 2:
                outs_d[s - 2].wait()
            acc_d[s % 2] = xld_f32[s % 2].astype(jnp.bfloat16) + recv_x[s % KS]
            outs_d[s] = pltpu.make_async_copy(
                acc_d.at[s % 2], out_ref.at[pl.ds(c_dir(s) * C, C), :],
                od_sem.at[s % 2])
            outs_d[s].start()

            if s >= 1:
                t = s - 1
                descs_y[t].wait_recv()
                loads_r[t].wait()
                if t + 1 < NS:
                    loads_r[t + 1] = load(my_col, c_rel(t + 1),
                                          xlr_f32.at[(t + 1) % 2],
                                          lr_sem.at[(t + 1) % 2])
                if t >= 2:
                    outs_r[t - 2].wait()
                acc_r[t % 2] = (xlr_f32[t % 2].astype(jnp.bfloat16)
                                + recv_y[t % KS])
                outs_r[t] = pltpu.make_async_copy(
                    acc_r.at[t % 2], out_ref.at[pl.ds(c_rel(t) * C, C), :],
                    or_sem.at[t % 2])
                outs_r[t].start()
                if t + KS < NS:
                    pl.semaphore_signal(credit_y, inc=1, device_id=ynbr,
                                        device_id_type=pl.DeviceIdType.MESH)

            if s >= CREDIT_LAG:
                u = s - CREDIT_LAG
                descs_y[u].wait_send()
                if u + KS < NS:
                    pl.semaphore_signal(credit_x, inc=1, device_id=peer,
                                        device_id_type=pl.DeviceIdType.MESH)

        t = NS - 1
        descs_y[t].wait_recv()
        loads_r[t].wait()
        outs_r[t - 2].wait()
        acc_r[t % 2] = xlr_f32[t % 2].astype(jnp.bfloat16) + recv_y[t % KS]
        outs_r[t] = pltpu.make_async_copy(
            acc_r.at[t % 2], out_ref.at[pl.ds(c_rel(t) * C, C), :],
            or_sem.at[t % 2])
        outs_r[t].start()

        for v in range(NS - SS, NS):
            descs_x[v].wait_send()
        for u in range(NS - CREDIT_LAG, NS):
            descs_y[u].wait_send()
        for w in range(NS - 2, NS):
            outs_d[w].wait()
            outs_r[w].wait()

    return pl.pallas_call(
        body,
        out_shape=jax.ShapeDtypeStruct((M, NH), jnp.bfloat16),
        in_specs=[pl.BlockSpec(memory_space=pl.ANY)],
        out_specs=pl.BlockSpec(memory_space=pl.ANY),
        scratch_shapes=[
            pltpu.VMEM((2, C, NH), jnp.float32),
            pltpu.VMEM((2, C, NH), jnp.float32),
            pltpu.VMEM((2, C, NH), jnp.float32),
            pltpu.VMEM((SS, C, NH), jnp.bfloat16),
            pltpu.VMEM((KS, C, NH), jnp.bfloat16),
            pltpu.VMEM((KS, C, NH), jnp.bfloat16),
            pltpu.VMEM((2, C, NH), jnp.bfloat16),
            pltpu.VMEM((2, C, NH), jnp.bfloat16),
            pltpu.SemaphoreType.DMA((2,)),
            pltpu.SemaphoreType.DMA((2,)),
            pltpu.SemaphoreType.DMA((2,)),
            pltpu.SemaphoreType.DMA((2,)),
            pltpu.SemaphoreType.DMA((2,)),
            pltpu.SemaphoreType.DMA((KS,)),
            pltpu.SemaphoreType.DMA((KS,)),
            pltpu.SemaphoreType.DMA((KS,)),
            pltpu.SemaphoreType.DMA((KS,)),
            pltpu.SemaphoreType.REGULAR,
            pltpu.SemaphoreType.REGULAR,
        ],
        compiler_params=pltpu.CompilerParams(
            collective_id=0, vmem_limit_bytes=48 * 1024 * 1024),
    )(x)


# device time: 212991 ns/iter; 2.3104x vs baseline; 1.0005x over previous
import jax
import jax.numpy as jnp
from jax import lax
from jax.experimental import pallas as pl
from jax.experimental.pallas import tpu as pltpu

M = 16384
N = 2048
NH = 1024
C = 128
NC = M // C
NS = NC // 2
KS = 8
SS = 4
CREDIT_LAG = 3


def kernel(x):
    def body(x_ref, out_ref, xs_f32, xld_f32, xlr_f32, send_buf, recv_x,
             recv_y, acc_d, acc_r, ls_sem, ld_sem, lr_sem, od_sem, or_sem,
             send_sem_x, recv_sem_x, send_sem_y, recv_sem_y,
             credit_x, credit_y):
        my_x = lax.axis_index("x")
        my_y = lax.axis_index("y")
        my_z = lax.axis_index("z")
        peer = (1 - my_x, my_y, my_z)
        ynbr = (my_x, 1 - my_y, my_z)
        my_col = my_x * NH
        peer_col = (1 - my_x) * NH

        def c_dir(s):
            return 2 * s + my_y

        def c_rel(s):
            return 2 * s + (1 - my_y)

        def load(cols, c, dst, sem):
            cp = pltpu.make_async_copy(
                x_ref.at[0, pl.ds(c * C, C), pl.ds(cols, NH)], dst, sem)
            cp.start()
            return cp

        def rdma_dir(s):
            return pltpu.make_async_remote_copy(
                src_ref=send_buf.at[s % SS], dst_ref=recv_x.at[s % KS],
                send_sem=send_sem_x.at[s % KS], recv_sem=recv_sem_x.at[s % KS],
                device_id=peer, device_id_type=pl.DeviceIdType.MESH)

        def rdma_rel(s):
            return pltpu.make_async_remote_copy(
                src_ref=recv_x.at[s % KS], dst_ref=recv_y.at[s % KS],
                send_sem=send_sem_y.at[s % KS], recv_sem=recv_sem_y.at[s % KS],
                device_id=ynbr, device_id_type=pl.DeviceIdType.MESH)

        descs_x = [None] * NS
        descs_y = [None] * NS
        loads_s = [None] * NS
        loads_d = [None] * NS
        loads_r = [None] * NS
        outs_d = [None] * NS
        outs_r = [None] * NS

        loads_s[0] = load(peer_col, c_dir(0), xs_f32.at[0], ls_sem.at[0])
        loads_s[1] = load(peer_col, c_dir(1), xs_f32.at[1], ls_sem.at[1])
        loads_d[0] = load(my_col, c_dir(0), xld_f32.at[0], ld_sem.at[0])
        loads_r[0] = load(my_col, c_rel(0), xlr_f32.at[0], lr_sem.at[0])

        barrier = pltpu.get_barrier_semaphore()
        for nbr in (peer, ynbr):
            pl.semaphore_signal(barrier, inc=1, device_id=nbr,
                                device_id_type=pl.DeviceIdType.MESH)
        pl.semaphore_wait(barrier, 2)

        for v in (0, 1, 2):
            loads_s[v].wait()
            send_buf[v % SS] = xs_f32[v % 2].astype(jnp.bfloat16)
            loads_s[v + 2] = load(peer_col, c_dir(v + 2), xs_f32.at[v % 2],
                                  ls_sem.at[v % 2])
            descs_x[v] = rdma_dir(v)
            descs_x[v].start()

        for s in range(NS):
            descs_x[s].wait_recv()
            if s >= KS:
                pl.semaphore_wait(credit_y, 1)
            descs_y[s] = rdma_rel(s)
            descs_y[s].start()

            v = s + 3
            if v < NS:
                loads_s[v].wait()
                if v >= SS:
                    descs_x[v - SS].wait_send()
                send_buf[v % SS] = xs_f32[v % 2].astype(jnp.bfloat16)
                if v + 2 < NS:
                    loads_s[v + 2] = load(peer_col, c_dir(v + 2),
                                          xs_f32.at[v % 2], ls_sem.at[v % 2])
                if v >= KS:
                    pl.semaphore_wait(credit_x, 1)
                descs_x[v] = rdma_dir(v)
                descs_x[v].start()

            loads_d[s].wait()
            if s + 1 < NS:
                loads_d[s + 1] = load(my_col, c_dir(s + 1),
                                      xld_f32.at[(s + 1) % 2],
                                      ld_sem.at[(s + 1) % 2])
            if s >= 2:
                outs_d[s - 2].wait()
            acc_d[s % 2] = xld_f32[s % 2].astype(jnp.bfloat16) + recv_x[s % KS]
            outs_d[s] = pltpu.make_async_copy(
                acc_d.at[s % 2], out_ref.at[pl.ds(c_dir(s) * C, C), :],
                od_sem.at[s % 2])
            outs_d[s].start()

            if s >= 1:
                t = s - 1
                descs_y[t].wait_recv()
                loads_r[t].wait()
                if t + 1 < NS:
                    loads_r[t + 1] = load(my_col, c_rel(t + 1),
                                          xlr_f32.at[(t + 1) % 2],
                                          lr_sem.at[(t + 1) % 2])
                if t >= 2:
                    outs_r[t - 2].wait()
                acc_r[t % 2] = (xlr_f32[t % 2].astype(jnp.bfloat16)
                                + recv_y[t % KS])
                outs_r[t] = pltpu.make_async_copy(
                    acc_r.at[t % 2], out_ref.at[pl.ds(c_rel(t) * C, C), :],
                    or_sem.at[t % 2])
                outs_r[t].start()
                if t + KS < NS:
                    pl.semaphore_signal(credit_y, inc=1, device_id=ynbr,
                                        device_id_type=pl.DeviceIdType.MESH)

            if s >= CREDIT_LAG:
                u = s - CREDIT_LAG
                descs_y[u].wait_send()
                if u + KS < NS:
                    pl.semaphore_signal(credit_x, inc=1, device_id=peer,
                                        device_id_type=pl.DeviceIdType.MESH)

        t = NS - 1
        descs_y[t].wait_recv()
        loads_r[t].wait()
        outs_r[t - 2].wait()
        acc_r[t % 2] = xlr_f32[t % 2].astype(jnp.bfloat16) + recv_y[t % KS]
        outs_r[t] = pltpu.make_async_copy(
            acc_r.at[t % 2], out_ref.at[pl.ds(c_rel(t) * C, C), :],
            or_sem.at[t % 2])
        outs_r[t].start()

        for v in range(NS - SS, NS):
            descs_x[v].wait_send()
        for u in range(NS - CREDIT_LAG, NS):
            descs_y[u].wait_send()
        for w in range(NS - 2, NS):
            outs_d[w].wait()
            outs_r[w].wait()

    return pl.pallas_call(
        body,
        out_shape=jax.ShapeDtypeStruct((M, NH), jnp.bfloat16),
        in_specs=[pl.BlockSpec(memory_space=pl.ANY)],
        out_specs=pl.BlockSpec(memory_space=pl.ANY),
        scratch_shapes=[
            pltpu.VMEM((2, C, NH), jnp.float32),
            pltpu.VMEM((2, C, NH), jnp.float32),
            pltpu.VMEM((2, C, NH), jnp.float32),
            pltpu.VMEM((SS, C, NH), jnp.bfloat16),
            pltpu.VMEM((KS, C, NH), jnp.bfloat16),
            pltpu.VMEM((KS, C, NH), jnp.bfloat16),
            pltpu.VMEM((2, C, NH), jnp.bfloat16),
            pltpu.VMEM((2, C, NH), jnp.bfloat16),
            pltpu.SemaphoreType.DMA((2,)),
            pltpu.SemaphoreType.DMA((2,)),
            pltpu.SemaphoreType.DMA((2,)),
            pltpu.SemaphoreType.DMA((2,)),
            pltpu.SemaphoreType.DMA((2,)),
            pltpu.SemaphoreType.DMA((KS,)),
            pltpu.SemaphoreType.DMA((KS,)),
            pltpu.SemaphoreType.DMA((KS,)),
            pltpu.SemaphoreType.DMA((KS,)),
            pltpu.SemaphoreType.REGULAR,
            pltpu.SemaphoreType.REGULAR,
        ],
        compiler_params=pltpu.CompilerParams(
            collective_id=0, vmem_limit_bytes=48 * 1024 * 1024),
    )(x)
